# Initial kernel scaffold; baseline (speedup 1.0000x reference)
import functools

import jax
import jax.numpy as jnp
from jax import lax
from jax.experimental import pallas as pl
from jax.experimental.pallas import tpu as pltpu
from jax.experimental.pallas import tpu_sc as plsc

N_NODES = 10000
N_EDGES = 320000
CH = 128

NC = 2
NS = 16
NW = NC * NS
E_PER_W = N_EDGES // NW
CHUNK = 80
N_CHUNKS = E_PER_W // CHUNK
ROWS_PER_TILE = N_NODES // NS
ZROWS = 125
NLANE = 16


def _sc_body(x_hbm, src_hbm, dst_hbm, w_hbm, out_hbm,
             src_v, dst_v, w_v, rows_v, acc, sem):
    c = lax.axis_index("c")
    s = lax.axis_index("s")
    wid = c * NS + s

    zero16 = jnp.zeros((NLANE,), jnp.float32)

    def zrow(i, carry):
        for k in range(CH // NLANE):
            rows_v[i, pl.ds(k * NLANE, NLANE)] = zero16
        return carry

    lax.fori_loop(0, ZROWS, zrow, 0)
    for j in range(ROWS_PER_TILE // ZROWS):
        r0 = s * ROWS_PER_TILE + j * ZROWS
        pltpu.sync_copy(rows_v.at[pl.ds(0, ZROWS)], acc.at[pl.ds(r0, ZROWS)])
    plsc.subcore_barrier()

    base0 = wid * E_PER_W

    def chunk_body(i, carry):
        base = base0 + i * CHUNK
        pltpu.sync_copy(src_hbm.at[pl.ds(base, CHUNK)], src_v)
        pltpu.sync_copy(dst_hbm.at[pl.ds(base, CHUNK)], dst_v)
        pltpu.sync_copy(w_hbm.at[pl.ds(base, CHUNK)], w_v)
        pltpu.async_copy(x_hbm.at[src_v], rows_v.at[pl.ds(0, CHUNK)], sem).wait()

        def scale(e, carry2):
            wv = plsc.load_gather(w_v, [jnp.full((NLANE,), e, jnp.int32)])
            for k in range(CH // NLANE):
                sl = pl.ds(k * NLANE, NLANE)
                rows_v[e, sl] = rows_v[e, sl] * wv
            return carry2

        lax.fori_loop(0, CHUNK, scale, 0)
        pltpu.sync_copy(rows_v.at[pl.ds(0, CHUNK)], acc.at[dst_v], add=True)
        return carry

    lax.fori_loop(0, N_CHUNKS, chunk_body, 0)
    plsc.subcore_barrier()

    for j in range(ROWS_PER_TILE // ZROWS):
        r0 = s * ROWS_PER_TILE + j * ZROWS
        pltpu.sync_copy(acc.at[pl.ds(r0, ZROWS)], out_hbm.at[c, pl.ds(r0, ZROWS)])


@jax.jit
def _sc_aggregate(x, src, dst, w):
    mesh = plsc.VectorSubcoreMesh(core_axis_name="c", subcore_axis_name="s")
    return pl.kernel(
        _sc_body,
        out_type=jax.ShapeDtypeStruct((NC, N_NODES, CH), jnp.float32),
        mesh=mesh,
        scratch_types=[
            pltpu.VMEM((CHUNK,), jnp.int32),
            pltpu.VMEM((CHUNK,), jnp.int32),
            pltpu.VMEM((CHUNK,), jnp.float32),
            pltpu.VMEM((ZROWS, CH), jnp.float32),
            pltpu.VMEM_SHARED((N_NODES, CH), jnp.float32),
            pltpu.SemaphoreType.DMA,
        ],
    )(x, src, dst, w)


def _mm_body(p_ref, w_ref, o_ref):
    s = p_ref[0] + p_ref[1]
    o_ref[...] = jnp.dot(s, w_ref[...], preferred_element_type=jnp.float32)


BLK = 1000


@jax.jit
def _combine_matmul(partials, W):
    return pl.pallas_call(
        _mm_body,
        grid=(N_NODES // BLK,),
        in_specs=[
            pl.BlockSpec((NC, BLK, CH), lambda i: (0, i, 0)),
            pl.BlockSpec((CH, CH), lambda i: (0, 0)),
        ],
        out_specs=pl.BlockSpec((BLK, CH), lambda i: (i, 0)),
        out_shape=jax.ShapeDtypeStruct((N_NODES, CH), jnp.float32),
    )(partials, W)


def kernel(x, W, edge_index, edge_weight):
    src = edge_index[0].astype(jnp.int32)
    dst = edge_index[1].astype(jnp.int32)
    partials = _sc_aggregate(x, src, dst, edge_weight)
    return _combine_matmul(partials, W)


# speedup vs baseline: 4.5317x; 4.5317x over previous
import functools

import jax
import jax.numpy as jnp
from jax import lax
from jax.experimental import pallas as pl
from jax.experimental.pallas import tpu as pltpu
from jax.experimental.pallas import tpu_sc as plsc

N_NODES = 10000
N_EDGES = 320000
CH = 128

NC = 2
NS = 16
NW = NC * NS
E_PER_W = N_EDGES // NW
CHUNK = 80
N_CHUNKS = E_PER_W // CHUNK
ZROWS = 200
NZBLOCKS = N_NODES // ZROWS
NLANE = 16


def _sc_body(x_hbm, src_hbm, dst_hbm, w_hbm, out_hbm,
             src_v, dst_v, w_v, rows_v, acc, sem):
    c = lax.axis_index("c")
    s = lax.axis_index("s")
    wid = c * NS + s

    zero16 = jnp.zeros((NLANE,), jnp.float32)

    def zrow(i, carry):
        for k in range(CH // NLANE):
            rows_v[i, pl.ds(k * NLANE, NLANE)] = zero16
        return carry

    lax.fori_loop(0, ZROWS, zrow, 0)
    for k in range((NZBLOCKS + NS - 1) // NS):
        b = s + NS * k
        r0 = pl.multiple_of(b * ZROWS, 8)
        if (NS * k) + NS <= NZBLOCKS:
            pltpu.sync_copy(rows_v, acc.at[pl.ds(r0, ZROWS)])
        else:
            @pl.when(b < NZBLOCKS)
            def _():
                pltpu.sync_copy(rows_v, acc.at[pl.ds(r0, ZROWS)])
    plsc.subcore_barrier()

    base0 = wid * E_PER_W

    def chunk_body(i, carry):
        base = base0 + i * CHUNK
        pltpu.sync_copy(src_hbm.at[pl.ds(base, CHUNK)], src_v)
        pltpu.sync_copy(dst_hbm.at[pl.ds(base, CHUNK)], dst_v)
        pltpu.sync_copy(w_hbm.at[pl.ds(base, CHUNK)], w_v)
        pltpu.async_copy(x_hbm.at[src_v], rows_v.at[pl.ds(0, CHUNK)], sem).wait()

        def scaleg(g, carry2):
            w16 = w_v[pl.ds(g * NLANE, NLANE)]
            for r in range(NLANE):
                wv = jnp.full((NLANE,), w16[r])
                e = g * NLANE + r
                for k in range(CH // NLANE):
                    sl = pl.ds(k * NLANE, NLANE)
                    rows_v[e, sl] = rows_v[e, sl] * wv
            return carry2

        lax.fori_loop(0, CHUNK // NLANE, scaleg, 0)
        pltpu.sync_copy(rows_v.at[pl.ds(0, CHUNK)], acc.at[dst_v], add=True)
        return carry

    lax.fori_loop(0, N_CHUNKS, chunk_body, 0)
    plsc.subcore_barrier()

    for k in range((NZBLOCKS + NS - 1) // NS):
        b = s + NS * k
        r0 = pl.multiple_of(b * ZROWS, 8)
        if (NS * k) + NS <= NZBLOCKS:
            pltpu.sync_copy(acc.at[pl.ds(r0, ZROWS)],
                            out_hbm.at[c, pl.ds(r0, ZROWS)])
        else:
            @pl.when(b < NZBLOCKS)
            def _():
                pltpu.sync_copy(acc.at[pl.ds(r0, ZROWS)],
                                out_hbm.at[c, pl.ds(r0, ZROWS)])


@jax.jit
def _sc_aggregate(x, src, dst, w):
    mesh = plsc.VectorSubcoreMesh(core_axis_name="c", subcore_axis_name="s")
    return pl.kernel(
        _sc_body,
        out_type=jax.ShapeDtypeStruct((NC, N_NODES, CH), jnp.float32),
        mesh=mesh,
        scratch_types=[
            pltpu.VMEM((CHUNK,), jnp.int32),
            pltpu.VMEM((CHUNK,), jnp.int32),
            pltpu.VMEM((CHUNK,), jnp.float32),
            pltpu.VMEM((ZROWS, CH), jnp.float32),
            pltpu.VMEM_SHARED((N_NODES, CH), jnp.float32),
            pltpu.SemaphoreType.DMA,
        ],
    )(x, src, dst, w)


def _mm_body(p_ref, w_ref, o_ref):
    s = p_ref[0] + p_ref[1]
    o_ref[...] = jnp.dot(s, w_ref[...], preferred_element_type=jnp.float32)


BLK = 1000


@jax.jit
def _combine_matmul(partials, W):
    return pl.pallas_call(
        _mm_body,
        grid=(N_NODES // BLK,),
        in_specs=[
            pl.BlockSpec((NC, BLK, CH), lambda i: (0, i, 0)),
            pl.BlockSpec((CH, CH), lambda i: (0, 0)),
        ],
        out_specs=pl.BlockSpec((BLK, CH), lambda i: (i, 0)),
        out_shape=jax.ShapeDtypeStruct((N_NODES, CH), jnp.float32),
    )(partials, W)


def kernel(x, W, edge_index, edge_weight):
    src = edge_index[0].astype(jnp.int32)
    dst = edge_index[1].astype(jnp.int32)
    partials = _sc_aggregate(x, src, dst, edge_weight)
    return _combine_matmul(partials, W)


# speedup vs baseline: 10.2231x; 2.2559x over previous
import jax
import jax.numpy as jnp
from jax import lax
from jax.experimental import pallas as pl
from jax.experimental.pallas import tpu as pltpu
from jax.experimental.pallas import tpu_sc as plsc

N_NODES = 10000
N_EDGES = 320000
CH = 128

NC = 2
NS = 16
NW = NC * NS
E_PER_W = N_EDGES // NW
CHUNK = 80
N_CHUNKS = E_PER_W // CHUNK
MBLK = 25
NMBLK = N_CHUNKS // MBLK
ZROWS = 80
NZBLOCKS = N_NODES // ZROWS
NLANE = 16


def _sc_body(x_hbm, srcm_hbm, dstm_hbm, wm_hbm, out_hbm,
             srcm, dstm, wm, rows_a, rows_b, acc, sem_a, sem_b):
    c = lax.axis_index("c")
    s = lax.axis_index("s")
    wid = c * NS + s

    zero16 = jnp.zeros((NLANE,), jnp.float32)

    def zrow(i, carry):
        for k in range(CH // NLANE):
            rows_a[i, pl.ds(k * NLANE, NLANE)] = zero16
        return carry

    lax.fori_loop(0, ZROWS, zrow, 0)
    for k in range((NZBLOCKS + NS - 1) // NS):
        b = s + NS * k
        r0 = pl.multiple_of(b * ZROWS, 8)
        if (NS * k) + NS <= NZBLOCKS:
            pltpu.sync_copy(rows_a, acc.at[pl.ds(r0, ZROWS)])
        else:
            @pl.when(b < NZBLOCKS)
            def _():
                pltpu.sync_copy(rows_a, acc.at[pl.ds(r0, ZROWS)])
    plsc.subcore_barrier()

    bufs = (rows_a, rows_b)
    sems = (sem_a, sem_b)

    def do_chunk(i, buf, sem, refill):
        pltpu.make_async_copy(x_hbm.at[srcm.at[i]], buf, sem).wait()

        def scaleg(g, carry2):
            w16 = wm[i, pl.ds(g * NLANE, NLANE)]
            for r in range(NLANE):
                wv = jnp.full((NLANE,), w16[r])
                for k in range(CH // NLANE):
                    sl = pl.ds(k * NLANE, NLANE)
                    buf[g * NLANE + r, sl] = buf[g * NLANE + r, sl] * wv
            return carry2

        lax.fori_loop(0, CHUNK // NLANE, scaleg, 0)
        pltpu.sync_copy(buf, acc.at[dstm.at[i]], add=True)
        if refill:
            @pl.when(i + 2 < MBLK)
            def _():
                pltpu.async_copy(x_hbm.at[srcm.at[i + 2]], buf, sem)

    for mb in range(NMBLK):
        pltpu.sync_copy(srcm_hbm.at[wid, mb], srcm)
        pltpu.sync_copy(dstm_hbm.at[wid, mb], dstm)
        pltpu.sync_copy(wm_hbm.at[wid, mb], wm)
        pltpu.async_copy(x_hbm.at[srcm.at[0]], rows_a, sem_a)
        pltpu.async_copy(x_hbm.at[srcm.at[1]], rows_b, sem_b)

        def pair_body(i2, carry):
            for b in range(2):
                do_chunk(i2 * 2 + b, bufs[b], sems[b], refill=True)
            return carry

        lax.fori_loop(0, MBLK // 2, pair_body, 0)
        if MBLK % 2:
            do_chunk(MBLK - 1, bufs[(MBLK - 1) % 2],
                     sems[(MBLK - 1) % 2], refill=False)
    plsc.subcore_barrier()

    for k in range((NZBLOCKS + NS - 1) // NS):
        b = s + NS * k
        r0 = pl.multiple_of(b * ZROWS, 8)
        if (NS * k) + NS <= NZBLOCKS:
            pltpu.sync_copy(acc.at[pl.ds(r0, ZROWS)],
                            out_hbm.at[c, pl.ds(r0, ZROWS)])
        else:
            @pl.when(b < NZBLOCKS)
            def _():
                pltpu.sync_copy(acc.at[pl.ds(r0, ZROWS)],
                                out_hbm.at[c, pl.ds(r0, ZROWS)])


@jax.jit
def _sc_aggregate(x, srcm, dstm, wm):
    mesh = plsc.VectorSubcoreMesh(core_axis_name="c", subcore_axis_name="s")
    return pl.kernel(
        _sc_body,
        out_type=jax.ShapeDtypeStruct((NC, N_NODES, CH), jnp.float32),
        mesh=mesh,
        scratch_types=[
            pltpu.VMEM((MBLK, CHUNK), jnp.int32),
            pltpu.VMEM((MBLK, CHUNK), jnp.int32),
            pltpu.VMEM((MBLK, CHUNK), jnp.float32),
            pltpu.VMEM((CHUNK, CH), jnp.float32),
            pltpu.VMEM((CHUNK, CH), jnp.float32),
            pltpu.VMEM_SHARED((N_NODES, CH), jnp.float32),
            pltpu.SemaphoreType.DMA,
            pltpu.SemaphoreType.DMA,
        ],
    )(x, srcm, dstm, wm)


def _mm_body(p_ref, w_ref, o_ref):
    s = p_ref[0] + p_ref[1]
    o_ref[...] = jnp.dot(s, w_ref[...], preferred_element_type=jnp.float32)


BLK = 1000


@jax.jit
def _combine_matmul(partials, W):
    return pl.pallas_call(
        _mm_body,
        grid=(N_NODES // BLK,),
        in_specs=[
            pl.BlockSpec((NC, BLK, CH), lambda i: (0, i, 0)),
            pl.BlockSpec((CH, CH), lambda i: (0, 0)),
        ],
        out_specs=pl.BlockSpec((BLK, CH), lambda i: (i, 0)),
        out_shape=jax.ShapeDtypeStruct((N_NODES, CH), jnp.float32),
    )(partials, W)


def kernel(x, W, edge_index, edge_weight):
    src = edge_index[0].astype(jnp.int32).reshape(NW, NMBLK, MBLK, CHUNK)
    dst = edge_index[1].astype(jnp.int32).reshape(NW, NMBLK, MBLK, CHUNK)
    w = edge_weight.reshape(NW, NMBLK, MBLK, CHUNK)
    partials = _sc_aggregate(x, src, dst, w)
    return _combine_matmul(partials, W)
